# baseline (speedup 1.0000x reference)
import jax
import jax.numpy as jnp
from jax.experimental import pallas as pl


def _gate_msg_call(rel, h_src, Wk1p, bk1, Wk2):
    E, ic = h_src.shape
    KC = Wk1p.shape[1]
    BLK = 4000

    def body(rel_ref, hs_ref, w1_ref, b1_ref, w2_ref, o_ref):
        hid = jnp.maximum(
            jnp.dot(rel_ref[...], w1_ref[...], preferred_element_type=jnp.float32)
            + b1_ref[...], 0.0)
        gate = jax.nn.sigmoid(
            jnp.dot(hid, w2_ref[...], preferred_element_type=jnp.float32))
        o_ref[...] = gate * hs_ref[...]

    return pl.pallas_call(
        body,
        grid=(E // BLK,),
        in_specs=[
            pl.BlockSpec((BLK, rel.shape[1]), lambda i: (i, 0)),
            pl.BlockSpec((BLK, ic), lambda i: (i, 0)),
            pl.BlockSpec((rel.shape[1], KC), lambda i: (0, 0)),
            pl.BlockSpec((1, KC), lambda i: (0, 0)),
            pl.BlockSpec((KC, ic), lambda i: (0, 0)),
        ],
        out_specs=pl.BlockSpec((BLK, ic), lambda i: (i, 0)),
        out_shape=jax.ShapeDtypeStruct((E, ic), jnp.float32),
    )(rel, h_src, Wk1p, bk1.reshape(1, KC), Wk2)


def _dense_tail_call(agg, h, Wlin, Wres, gamma, beta):
    n, ic = agg.shape
    oc = Wlin.shape[1]
    BLK = 1000

    def body(a_ref, h_ref, wl_ref, wr_ref, g_ref, b_ref, o_ref):
        out = jnp.dot(a_ref[...], wl_ref[...],
                      preferred_element_type=jnp.float32) * g_ref[...] + b_ref[...]
        out = out + jnp.dot(h_ref[...], wr_ref[...],
                            preferred_element_type=jnp.float32)
        o_ref[...] = jnp.maximum(out, 0.0)

    return pl.pallas_call(
        body,
        grid=(n // BLK,),
        in_specs=[
            pl.BlockSpec((BLK, ic), lambda i: (i, 0)),
            pl.BlockSpec((BLK, ic), lambda i: (i, 0)),
            pl.BlockSpec((ic, oc), lambda i: (0, 0)),
            pl.BlockSpec((ic, oc), lambda i: (0, 0)),
            pl.BlockSpec((1, oc), lambda i: (0, 0)),
            pl.BlockSpec((1, oc), lambda i: (0, 0)),
        ],
        out_specs=pl.BlockSpec((BLK, oc), lambda i: (i, 0)),
        out_shape=jax.ShapeDtypeStruct((n, oc), jnp.float32),
    )(agg, h, Wlin, Wres, gamma.reshape(1, oc), beta.reshape(1, oc))


def _heads_call(xg, seq_emb, domain, Wps, Wpq, Wpd, Wss, Wcp,
                Wc1a, gc1a, bc1a, Wc2a, Wc1b, gc1b, bc1b, Wc2b):
    B = xg.shape[0]
    NCLS = Wc2a.shape[1]
    PROJ = Wps.shape[1]

    def body(xg_ref, se_ref, dom_ref, wps, wpq, wpd, wss, wcp,
             w1a, g1a, b1a, w2a, w1b, g1b, b1b, w2b, o_ref):
        f32 = jnp.float32
        struct = jnp.dot(xg_ref[...], wps[...], preferred_element_type=f32)
        seqf = jnp.dot(se_ref[...], wpq[...], preferred_element_type=f32)
        domf = jnp.dot(dom_ref[...], wpd[...], preferred_element_type=f32)
        f_inc = jnp.dot(jnp.concatenate([struct, seqf], axis=-1), wss[...],
                        preferred_element_type=f32)
        inc = jnp.dot(
            jnp.maximum(jnp.dot(f_inc, w1b[...], preferred_element_type=f32)
                        * g1b[...] + b1b[...], 0.0),
            w2b[...], preferred_element_type=f32)
        f_cmp = jnp.dot(jnp.concatenate([struct, seqf, domf], axis=-1), wcp[...],
                        preferred_element_type=f32)
        comp = jnp.dot(
            jnp.maximum(jnp.dot(f_cmp, w1a[...], preferred_element_type=f32)
                        * g1a[...] + b1a[...], 0.0),
            w2a[...], preferred_element_type=f32)
        didx = jnp.sum(dom_ref[...], axis=-1, keepdims=True) != 0.0
        o_ref[...] = jnp.where(didx, comp, inc)

    return pl.pallas_call(
        body,
        out_shape=jax.ShapeDtypeStruct((B, NCLS), jnp.float32),
    )(xg, seq_emb, domain, Wps, Wpq, Wpd, Wss, Wcp,
      Wc1a, gc1a.reshape(1, PROJ), bc1a.reshape(1, PROJ), Wc2a,
      Wc1b, gc1b.reshape(1, PROJ), bc1b.reshape(1, PROJ), Wc2b)


def _run_block(h, pos, seq, ori, src, dst, n, Wk1, bk1, Wk2, Wlin, Wres,
               gamma, beta):
    rel = jnp.concatenate([
        pos[dst] - pos[src],
        (seq[dst] - seq[src])[:, None],
        jnp.sum(ori[src] * ori[dst], axis=-1, keepdims=True),
        jnp.zeros((src.shape[0], 3), jnp.float32),
    ], axis=-1)
    Wk1p = jnp.concatenate([Wk1, jnp.zeros((3, Wk1.shape[1]), jnp.float32)],
                           axis=0)
    msg = _gate_msg_call(rel, h[src], Wk1p, bk1, Wk2)
    agg = jax.ops.segment_sum(msg, dst, num_segments=n)
    return _dense_tail_call(agg, h, Wlin, Wres, gamma, beta)


def kernel(x, pos, seq, ori, batch, edge_index1, edge_index2, seq_emb, domain,
           emb,
           b0_Wk1, b0_bk1, b0_Wk2, b0_Wlin, b0_Wres, b0_gamma, b0_beta,
           b1_Wk1, b1_bk1, b1_Wk2, b1_Wlin, b1_Wres, b1_gamma, b1_beta,
           b2_Wk1, b2_bk1, b2_Wk2, b2_Wlin, b2_Wres, b2_gamma, b2_beta,
           b3_Wk1, b3_bk1, b3_Wk2, b3_Wlin, b3_Wres, b3_gamma, b3_beta,
           Wps, Wpq, Wpd, Wss, Wcp,
           Wc1a, gc1a, bc1a, Wc2a,
           Wc1b, gc1b, bc1b, Wc2b):
    N = pos.shape[0]
    B = seq_emb.shape[0]
    h = emb[x]
    s1, t1 = edge_index1[0], edge_index1[1]
    s2, t2 = edge_index2[0], edge_index2[1]

    h = _run_block(h, pos, seq, ori, s1, t1, N,
                   b0_Wk1, b0_bk1, b0_Wk2, b0_Wlin, b0_Wres, b0_gamma, b0_beta)
    h = _run_block(h, pos, seq, ori, s1, t1, N,
                   b1_Wk1, b1_bk1, b1_Wk2, b1_Wlin, b1_Wres, b1_gamma, b1_beta)

    h = 0.5 * (h[0::2] + h[1::2])
    pos = 0.5 * (pos[0::2] + pos[1::2])
    seq = 0.5 * (seq[0::2] + seq[1::2])
    ori = 0.5 * (ori[0::2] + ori[1::2])
    batch = batch[0::2]
    N2 = N // 2

    h = _run_block(h, pos, seq, ori, s2, t2, N2,
                   b2_Wk1, b2_bk1, b2_Wk2, b2_Wlin, b2_Wres, b2_gamma, b2_beta)
    h = _run_block(h, pos, seq, ori, s2, t2, N2,
                   b3_Wk1, b3_bk1, b3_Wk2, b3_Wlin, b3_Wres, b3_gamma, b3_beta)

    sums = jax.ops.segment_sum(h, batch, num_segments=B)
    cnt = jax.ops.segment_sum(jnp.ones((N2, 1), jnp.float32), batch,
                              num_segments=B)
    xg = sums / jnp.maximum(cnt, 1.0)

    return _heads_call(xg, seq_emb, domain, Wps, Wpq, Wpd, Wss, Wcp,
                       Wc1a, gc1a, bc1a, Wc2a, Wc1b, gc1b, bc1b, Wc2b)


# speedup vs baseline: 2.7328x; 2.7328x over previous
import functools

import jax
import jax.numpy as jnp
from jax import lax
from jax.experimental import pallas as pl
from jax.experimental.pallas import tpu as pltpu
from jax.experimental.pallas import tpu_sc as plsc

_SUB = 40
_CKE = 320
_CKZ = 625


def _gate_msg_call(gs, gd, hs, Wk1p, bk1, Wk2, nsplit):
    E, ic = hs.shape
    G = gs.shape[1]
    KC = Wk1p.shape[1]
    BLK = 4000
    icH = ic // nsplit

    def body(gs_ref, gd_ref, hs_ref, w1_ref, b1_ref, w2_ref, o_ref):
        a = gs_ref[...]
        b = gd_ref[...]
        rel = jnp.concatenate([
            b[:, 0:4] - a[:, 0:4],
            jnp.sum(a[:, 4:7] * b[:, 4:7], axis=1, keepdims=True),
            jnp.zeros((BLK, 3), jnp.float32),
        ], axis=-1)
        hid = jnp.maximum(
            jnp.dot(rel, w1_ref[...], preferred_element_type=jnp.float32)
            + b1_ref[...], 0.0)
        gate = jax.nn.sigmoid(
            jnp.dot(hid, w2_ref[...], preferred_element_type=jnp.float32))
        m = gate * hs_ref[...]
        if nsplit == 1:
            o_ref[...] = m[None]
        else:
            o_ref[...] = jnp.stack(
                [m[:, k * icH:(k + 1) * icH] for k in range(nsplit)], axis=0)

    return pl.pallas_call(
        body,
        grid=(E // BLK,),
        in_specs=[
            pl.BlockSpec((BLK, G), lambda i: (i, 0)),
            pl.BlockSpec((BLK, G), lambda i: (i, 0)),
            pl.BlockSpec((BLK, ic), lambda i: (i, 0)),
            pl.BlockSpec((8, KC), lambda i: (0, 0)),
            pl.BlockSpec((1, KC), lambda i: (0, 0)),
            pl.BlockSpec((KC, ic), lambda i: (0, 0)),
        ],
        out_specs=pl.BlockSpec((nsplit, BLK, icH), lambda i: (0, i, 0)),
        out_shape=jax.ShapeDtypeStruct((nsplit, E, icH), jnp.float32),
    )(gs, gd, hs, Wk1p, bk1.reshape(1, KC), Wk2)


def _sc_segment_sum(msgS, dst2, n, mode):
    nparts, E, icH = msgS.shape
    npass = 1 if mode == "edge" else nparts // 2
    nout = max(nparts, 2)
    nrow = _CKE // _SUB
    nchunks = E // _CKE if mode == "channel" else E // 2 // _CKE
    iters = -(-nchunks // 16)
    nzc = n // _CKZ
    zloops = (nzc + 15) // 16
    mesh = plsc.VectorSubcoreMesh(core_axis_name="c", subcore_axis_name="s")

    @functools.partial(
        pl.kernel, mesh=mesh,
        compiler_params=pltpu.CompilerParams(use_tc_tiling_on_sc=False),
        out_type=jax.ShapeDtypeStruct((nout, n, icH), jnp.float32),
        scratch_types=[
            pltpu.VMEM((nrow, _SUB), jnp.int32),
            pltpu.VMEM((_CKE, icH), jnp.float32),
            pltpu.VMEM((_CKZ, icH), jnp.float32),
            pltpu.VMEM_SHARED((n, icH), jnp.float32),
        ],
    )
    def k(msg_hbm, dst_hbm, out_hbm, idx_v, rows_v, zbuf_v, agg_sh):
        c = lax.axis_index("c")
        s = lax.axis_index("s")
        cbase = 0 if mode == "channel" else c * nchunks
        nv = icH // 16
        zero16 = jnp.zeros((16,), jnp.float32)

        for p in range(npass):
            def zrow(r, _):
                for q in range(nv):
                    zbuf_v[r, pl.ds(q * 16, 16)] = zero16
                return 0
            lax.fori_loop(0, _CKZ, zrow, 0)
            for i in range(zloops):
                ch = i * 16 + s
                @pl.when(ch < nzc)
                def _():
                    pltpu.sync_copy(zbuf_v, agg_sh.at[pl.ds(ch * _CKZ, _CKZ)])
            plsc.subcore_barrier()

            part = c * npass + p if mode == "channel" else 0

            def body(j, _):
                ci = j * 16 + s
                @pl.when(ci < nchunks)
                def _():
                    cc = cbase + ci
                    pltpu.sync_copy(dst_hbm.at[pl.ds(cc * nrow, nrow)], idx_v)
                    pltpu.sync_copy(msg_hbm.at[part, pl.ds(cc * _CKE, _CKE)],
                                    rows_v)
                    for q in range(nrow):
                        pltpu.sync_copy(rows_v.at[pl.ds(q * _SUB, _SUB)],
                                        agg_sh.at[idx_v.at[q]], add=True)
                return 0

            lax.fori_loop(0, iters, body, 0)
            plsc.subcore_barrier()
            outp = part if mode == "channel" else c
            for i in range(zloops):
                ch = i * 16 + s
                @pl.when(ch < nzc)
                def _():
                    pltpu.sync_copy(agg_sh.at[pl.ds(ch * _CKZ, _CKZ)], zbuf_v)
                    pltpu.sync_copy(zbuf_v,
                                    out_hbm.at[outp, pl.ds(ch * _CKZ, _CKZ)])
            if p + 1 < npass:
                plsc.subcore_barrier()

    return k(msgS, dst2)


def _sc_edge_gather(geom, h, src, dst):
    E = src.shape[0]
    ic = h.shape[1]
    G = geom.shape[1]
    nchunks = E // _CKE
    iters = -(-nchunks // 32)
    mesh = plsc.VectorSubcoreMesh(core_axis_name="c", subcore_axis_name="s")

    @functools.partial(
        pl.kernel, mesh=mesh,
        compiler_params=pltpu.CompilerParams(use_tc_tiling_on_sc=False),
        out_type=(jax.ShapeDtypeStruct((E, G), jnp.float32),
                  jax.ShapeDtypeStruct((E, G), jnp.float32),
                  jax.ShapeDtypeStruct((E, ic), jnp.float32)),
        scratch_types=[
            pltpu.VMEM((_CKE,), jnp.int32),
            pltpu.VMEM((_CKE,), jnp.int32),
            pltpu.VMEM((_CKE, G), jnp.float32),
            pltpu.VMEM((_CKE, G), jnp.float32),
            pltpu.VMEM((_CKE, ic), jnp.float32),
            pltpu.SemaphoreType.DMA,
            pltpu.SemaphoreType.DMA,
            pltpu.SemaphoreType.DMA,
        ],
    )
    def k(geom_hbm, h_hbm, src_hbm, dst_hbm, gs_out, gd_out, hs_out,
          sidx_v, didx_v, gs_v, gd_v, hs_v, sem0, sem1, sem2):
        c = lax.axis_index("c")
        s = lax.axis_index("s")
        wid = s * 2 + c

        def body(j, _):
            ci = j * 32 + wid
            @pl.when(ci < nchunks)
            def _():
                e0 = ci * _CKE
                pltpu.sync_copy(src_hbm.at[pl.ds(e0, _CKE)], sidx_v)
                pltpu.sync_copy(dst_hbm.at[pl.ds(e0, _CKE)], didx_v)
                d0 = pltpu.async_copy(geom_hbm.at[sidx_v], gs_v, sem0)
                d1 = pltpu.async_copy(geom_hbm.at[didx_v], gd_v, sem1)
                d2 = pltpu.async_copy(h_hbm.at[sidx_v], hs_v, sem2)
                d0.wait()
                d1.wait()
                d2.wait()
                pltpu.sync_copy(gs_v, gs_out.at[pl.ds(e0, _CKE)])
                pltpu.sync_copy(gd_v, gd_out.at[pl.ds(e0, _CKE)])
                pltpu.sync_copy(hs_v, hs_out.at[pl.ds(e0, _CKE)])
            return 0

        lax.fori_loop(0, iters, body, 0)

    return k(geom, h, src, dst)


def _dense_tail_call(agg, h, Wlin, Wres, gamma, beta):
    n, ic = agg.shape
    oc = Wlin.shape[1]
    BLK = 1000

    def body(a_ref, h_ref, wl_ref, wr_ref, g_ref, b_ref, o_ref):
        out = jnp.dot(a_ref[...], wl_ref[...],
                      preferred_element_type=jnp.float32) * g_ref[...] + b_ref[...]
        out = out + jnp.dot(h_ref[...], wr_ref[...],
                            preferred_element_type=jnp.float32)
        o_ref[...] = jnp.maximum(out, 0.0)

    return pl.pallas_call(
        body,
        grid=(n // BLK,),
        in_specs=[
            pl.BlockSpec((BLK, ic), lambda i: (i, 0)),
            pl.BlockSpec((BLK, ic), lambda i: (i, 0)),
            pl.BlockSpec((ic, oc), lambda i: (0, 0)),
            pl.BlockSpec((ic, oc), lambda i: (0, 0)),
            pl.BlockSpec((1, oc), lambda i: (0, 0)),
            pl.BlockSpec((1, oc), lambda i: (0, 0)),
        ],
        out_specs=pl.BlockSpec((BLK, oc), lambda i: (i, 0)),
        out_shape=jax.ShapeDtypeStruct((n, oc), jnp.float32),
    )(agg, h, Wlin, Wres, gamma.reshape(1, oc), beta.reshape(1, oc))


def _heads_call(xg, seq_emb, domain, Wps, Wpq, Wpd, Wss, Wcp,
                Wc1a, gc1a, bc1a, Wc2a, Wc1b, gc1b, bc1b, Wc2b):
    B = xg.shape[0]
    NCLS = Wc2a.shape[1]
    PROJ = Wps.shape[1]

    def body(xg_ref, se_ref, dom_ref, wps, wpq, wpd, wss, wcp,
             w1a, g1a, b1a, w2a, w1b, g1b, b1b, w2b, o_ref):
        f32 = jnp.float32
        struct = jnp.dot(xg_ref[...], wps[...], preferred_element_type=f32)
        seqf = jnp.dot(se_ref[...], wpq[...], preferred_element_type=f32)
        domf = jnp.dot(dom_ref[...], wpd[...], preferred_element_type=f32)
        f_inc = jnp.dot(jnp.concatenate([struct, seqf], axis=-1), wss[...],
                        preferred_element_type=f32)
        inc = jnp.dot(
            jnp.maximum(jnp.dot(f_inc, w1b[...], preferred_element_type=f32)
                        * g1b[...] + b1b[...], 0.0),
            w2b[...], preferred_element_type=f32)
        f_cmp = jnp.dot(jnp.concatenate([struct, seqf, domf], axis=-1), wcp[...],
                        preferred_element_type=f32)
        comp = jnp.dot(
            jnp.maximum(jnp.dot(f_cmp, w1a[...], preferred_element_type=f32)
                        * g1a[...] + b1a[...], 0.0),
            w2a[...], preferred_element_type=f32)
        didx = jnp.sum(dom_ref[...], axis=-1, keepdims=True) != 0.0
        o_ref[...] = jnp.where(didx, comp, inc)

    return pl.pallas_call(
        body,
        out_shape=jax.ShapeDtypeStruct((B, NCLS), jnp.float32),
    )(xg, seq_emb, domain, Wps, Wpq, Wpd, Wss, Wcp,
      Wc1a, gc1a.reshape(1, PROJ), bc1a.reshape(1, PROJ), Wc2a,
      Wc1b, gc1b.reshape(1, PROJ), bc1b.reshape(1, PROJ), Wc2b)


_USE_SC_GATHER = False


def _run_block(h, geom, src, dst, dst2, n, mode, nsplit, Wk1, bk1,
               Wk2, Wlin, Wres, gamma, beta):
    Wk1p = jnp.concatenate([Wk1, jnp.zeros((3, Wk1.shape[1]), jnp.float32)],
                           axis=0)
    if _USE_SC_GATHER:
        gs, gd, hs = _sc_edge_gather(geom, h, src, dst)
    else:
        gs, gd, hs = geom[src], geom[dst], h[src]
    msgS = _gate_msg_call(gs, gd, hs, Wk1p, bk1, Wk2, nsplit)
    aggS = _sc_segment_sum(msgS, dst2, n, mode)
    if mode == "channel":
        agg = jnp.concatenate([aggS[k] for k in range(nsplit)], axis=-1)
    else:
        agg = aggS[0] + aggS[1]
    return _dense_tail_call(agg, h, Wlin, Wres, gamma, beta)


def kernel(x, pos, seq, ori, batch, edge_index1, edge_index2, seq_emb, domain,
           emb,
           b0_Wk1, b0_bk1, b0_Wk2, b0_Wlin, b0_Wres, b0_gamma, b0_beta,
           b1_Wk1, b1_bk1, b1_Wk2, b1_Wlin, b1_Wres, b1_gamma, b1_beta,
           b2_Wk1, b2_bk1, b2_Wk2, b2_Wlin, b2_Wres, b2_gamma, b2_beta,
           b3_Wk1, b3_bk1, b3_Wk2, b3_Wlin, b3_Wres, b3_gamma, b3_beta,
           Wps, Wpq, Wpd, Wss, Wcp,
           Wc1a, gc1a, bc1a, Wc2a,
           Wc1b, gc1b, bc1b, Wc2b):
    N = pos.shape[0]
    B = seq_emb.shape[0]
    h = emb[x]
    s1, t1 = edge_index1[0], edge_index1[1]
    s2, t2 = edge_index2[0], edge_index2[1]
    t1b = t1.reshape(-1, _SUB)
    t2b = t2.reshape(-1, _SUB)
    geom = jnp.concatenate([pos, seq[:, None], ori,
                            jnp.zeros((N, 1), jnp.float32)], axis=-1)

    h = _run_block(h, geom, s1, t1, t1b, N, "edge", 1,
                   b0_Wk1, b0_bk1, b0_Wk2, b0_Wlin, b0_Wres, b0_gamma, b0_beta)
    h = _run_block(h, geom, s1, t1, t1b, N, "channel", 2,
                   b1_Wk1, b1_bk1, b1_Wk2, b1_Wlin, b1_Wres, b1_gamma, b1_beta)

    h = 0.5 * (h[0::2] + h[1::2])
    geom = 0.5 * (geom[0::2] + geom[1::2])
    batch = batch[0::2]
    N2 = N // 2

    h = _run_block(h, geom, s2, t2, t2b, N2, "channel", 2,
                   b2_Wk1, b2_bk1, b2_Wk2, b2_Wlin, b2_Wres, b2_gamma, b2_beta)
    h = _run_block(h, geom, s2, t2, t2b, N2, "channel", 2,
                   b3_Wk1, b3_bk1, b3_Wk2, b3_Wlin, b3_Wres, b3_gamma, b3_beta)

    sums = jax.ops.segment_sum(h, batch, num_segments=B)
    cnt = jax.ops.segment_sum(jnp.ones((N2, 1), jnp.float32), batch,
                              num_segments=B)
    xg = sums / jnp.maximum(cnt, 1.0)

    return _heads_call(xg, seq_emb, domain, Wps, Wpq, Wpd, Wss, Wcp,
                       Wc1a, gc1a, bc1a, Wc2a, Wc1b, gc1b, bc1b, Wc2b)
